# baseline (speedup 1.0000x reference)
import jax
import jax.numpy as jnp
from jax import lax
from jax.experimental import pallas as pl
from jax.experimental.pallas import tpu as pltpu
from jax.experimental.pallas import tpu_sc as plsc

VOCAB = 1_000_000
D = 64
T = 200
B = 4096
FLAT = B * T
NC = 2
NS = 16
NW = NC * NS
PER_W = FLAT // NW
CHUNK = 512
NCHUNK = PER_W // CHUNK
SUB = 128
NSUB = CHUNK // SUB
PROWS = 704
LANES = 16
KD = D // LANES


def _body(idx_hbm, tok_hbm, pos_hbm, out_hbm, idx_v, rows_v, pat_v, sem):
    cid = lax.axis_index("c")
    sid = lax.axis_index("s")
    wid = sid * NC + cid

    pltpu.sync_copy(pos_hbm, pat_v.at[pl.ds(0, T)])
    pltpu.sync_copy(pos_hbm, pat_v.at[pl.ds(T, T)])
    pltpu.sync_copy(pos_hbm, pat_v.at[pl.ds(2 * T, T)])
    pltpu.sync_copy(pos_hbm.at[pl.ds(0, PROWS - 3 * T)], pat_v.at[pl.ds(3 * T, PROWS - 3 * T)])

    def chunk_body(ci, _):
        base = wid * PER_W + ci * CHUNK
        t0 = lax.rem(base, T)

        pltpu.sync_copy(idx_hbm.at[pl.ds(base, CHUNK)], idx_v)

        copies = [
            pltpu.async_copy(
                tok_hbm.at[idx_v.at[pl.ds(j * SUB, SUB)]],
                rows_v.at[pl.ds(j * SUB, SUB)],
                sem,
            )
            for j in range(NSUB)
        ]
        for cp in copies:
            cp.wait()

        def r_body(r, _):
            for u in range(4):
                for k in range(KD):
                    p = pat_v[t0 + r + u, pl.ds(k * LANES, LANES)]
                    plsc.addupdate(rows_v.at[r + u, pl.ds(k * LANES, LANES)], p)
            return ()

        lax.fori_loop(0, CHUNK // 4, lambda i, c: r_body(i * 4, c), ())

        pltpu.sync_copy(rows_v, out_hbm.at[pl.ds(base, CHUNK)])
        return ()

    lax.fori_loop(0, NCHUNK, chunk_body, ())


@jax.jit
def _embed(idx1d, tok, pos):
    mesh = plsc.VectorSubcoreMesh(core_axis_name="c", subcore_axis_name="s")
    f = pl.kernel(
        _body,
        mesh=mesh,
        out_type=jax.ShapeDtypeStruct((FLAT, D), jnp.float32),
        scratch_types=[
            pltpu.VMEM((CHUNK,), jnp.int32),
            pltpu.VMEM((CHUNK, D), jnp.float32),
            pltpu.VMEM((PROWS, D), jnp.float32),
            pltpu.SemaphoreType.DMA,
        ],
        compiler_params=pltpu.CompilerParams(use_tc_tiling_on_sc=False),
    )
    return f(idx1d, tok, pos)


def kernel(idx, token_embedding_table, position_embedding_table):
    idx1d = idx.astype(jnp.int32).reshape(FLAT)
    out = _embed(idx1d, token_embedding_table, position_embedding_table)
    return out.reshape(B, T, D)


# speedup vs baseline: 1.2973x; 1.2973x over previous
import jax
import jax.numpy as jnp
from jax import lax
from jax.experimental import pallas as pl
from jax.experimental.pallas import tpu as pltpu
from jax.experimental.pallas import tpu_sc as plsc

VOCAB = 1_000_000
D = 64
T = 200
B = 4096
FLAT = B * T
NC = 2
NS = 16
NW = NC * NS
PER_W = FLAT // NW
CHUNK = 512
NCHUNK = PER_W // CHUNK
SUB = 128
NSUB = CHUNK // SUB
PROWS = 704
LANES = 16
KD = D // LANES


def _body(idx_hbm, tok_hbm, pos_hbm, out_hbm,
          idx0, idx1, rows0, rows1, pat_v, sg0, sg1, so0, so1):
    cid = lax.axis_index("c")
    sid = lax.axis_index("s")
    wid = sid * NC + cid
    w0 = wid * PER_W

    pltpu.sync_copy(pos_hbm, pat_v.at[pl.ds(0, T)])
    pltpu.sync_copy(pos_hbm, pat_v.at[pl.ds(T, T)])
    pltpu.sync_copy(pos_hbm, pat_v.at[pl.ds(2 * T, T)])
    pltpu.sync_copy(pos_hbm.at[pl.ds(0, PROWS - 3 * T)],
                    pat_v.at[pl.ds(3 * T, PROWS - 3 * T)])

    def gather_start(ci, idxb, rowsb, sem):
        pltpu.sync_copy(idx_hbm.at[pl.ds(w0 + ci * CHUNK, CHUNK)], idxb)
        for j in range(NSUB):
            pltpu.async_copy(tok_hbm.at[idxb.at[pl.ds(j * SUB, SUB)]],
                             rowsb.at[pl.ds(j * SUB, SUB)], sem)

    def gather_wait(idxb, rowsb, sem):
        pltpu.make_async_copy(tok_hbm.at[pl.ds(0, CHUNK)], rowsb, sem).wait()

    def add_pos(ci, rowsb):
        t0 = lax.rem(w0 + ci * CHUNK, T)

        @plsc.parallel_loop(0, CHUNK, step=1, unroll=8)
        def _(r):
            for k in range(KD):
                p = pat_v[t0 + r, pl.ds(k * LANES, LANES)]
                plsc.addupdate(rowsb.at[r, pl.ds(k * LANES, LANES)], p)

    def out_start(ci, rowsb, sem):
        pltpu.async_copy(rowsb, out_hbm.at[pl.ds(w0 + ci * CHUNK, CHUNK)], sem)

    def out_wait(ci, rowsb, sem):
        pltpu.make_async_copy(
            rowsb, out_hbm.at[pl.ds(w0 + ci * CHUNK, CHUNK)], sem).wait()

    gather_start(0, idx0, rows0, sg0)
    NOUT = NCHUNK // 2

    def g_body(g, _):
        c0 = 2 * g

        @pl.when(g > 0)
        def _():
            out_wait(c0 - 1, rows1, so1)

        gather_start(c0 + 1, idx1, rows1, sg1)
        gather_wait(idx0, rows0, sg0)
        add_pos(c0, rows0)
        out_start(c0, rows0, so0)

        @pl.when(g < NOUT - 1)
        def _():
            out_wait(c0, rows0, so0)
            gather_start(c0 + 2, idx0, rows0, sg0)

        gather_wait(idx1, rows1, sg1)
        add_pos(c0 + 1, rows1)
        out_start(c0 + 1, rows1, so1)
        return ()

    lax.fori_loop(0, NOUT, g_body, ())
    out_wait(NCHUNK - 2, rows0, so0)
    out_wait(NCHUNK - 1, rows1, so1)


@jax.jit
def _embed(idx1d, tok, pos):
    mesh = plsc.VectorSubcoreMesh(core_axis_name="c", subcore_axis_name="s")
    f = pl.kernel(
        _body,
        mesh=mesh,
        out_type=jax.ShapeDtypeStruct((FLAT, D), jnp.float32),
        scratch_types=[
            pltpu.VMEM((CHUNK,), jnp.int32),
            pltpu.VMEM((CHUNK,), jnp.int32),
            pltpu.VMEM((CHUNK, D), jnp.float32),
            pltpu.VMEM((CHUNK, D), jnp.float32),
            pltpu.VMEM((PROWS, D), jnp.float32),
            pltpu.SemaphoreType.DMA,
            pltpu.SemaphoreType.DMA,
            pltpu.SemaphoreType.DMA,
            pltpu.SemaphoreType.DMA,
        ],
        compiler_params=pltpu.CompilerParams(use_tc_tiling_on_sc=False),
    )
    return f(idx1d, tok, pos)


def kernel(idx, token_embedding_table, position_embedding_table):
    idx1d = idx.astype(jnp.int32).reshape(FLAT)
    out = _embed(idx1d, token_embedding_table, position_embedding_table)
    return out.reshape(B, T, D)


# speedup vs baseline: 1.3400x; 1.0330x over previous
import jax
import jax.numpy as jnp
from jax import lax
from jax.experimental import pallas as pl
from jax.experimental.pallas import tpu as pltpu
from jax.experimental.pallas import tpu_sc as plsc

VOCAB = 1_000_000
D = 64
T = 200
B = 4096
FLAT = B * T
NC = 2
NS = 16
NW = NC * NS
PER_W = FLAT // NW
CHUNK = 128
NCHUNK = PER_W // CHUNK
NBUF = 4
NGRP = NCHUNK // NBUF
LANES = 16
KD = D // LANES
VPC = CHUNK // LANES


def _body(idx_hbm, tok_hbm, pos_hbm, out_hbm,
          i0, i1, i2, i3, rows_v, ob0, ob1, pat_v,
          sg0, sg1, sg2, sg3, so0, so1):
    cid = lax.axis_index("c")
    sid = lax.axis_index("s")
    wid = sid * NC + cid
    w0 = wid * PER_W
    idxb = [i0, i1, i2, i3]
    sg = [sg0, sg1, sg2, sg3]
    outb = [ob0, ob1]
    so = [so0, so1]

    pltpu.sync_copy(pos_hbm, pat_v)

    def gather_start(ci, b):
        base = w0 + ci * CHUNK
        pltpu.sync_copy(idx_hbm.at[pl.ds(base, CHUNK)], idxb[b])
        pltpu.async_copy(tok_hbm.at[idxb[b]], rows_v.at[b], sg[b])

    def gather_wait(b):
        pltpu.make_async_copy(tok_hbm.at[pl.ds(0, CHUNK)], rows_v.at[b], sg[b]).wait()

    def out_start(ci, ob):
        base = w0 + ci * CHUNK
        pltpu.async_copy(outb[ob], out_hbm.at[pl.ds(base, CHUNK)], so[ob])

    def out_wait(ci, ob):
        base = w0 + ci * CHUNK
        pltpu.make_async_copy(outb[ob], out_hbm.at[pl.ds(base, CHUNK)], so[ob]).wait()

    def extract_add(ci, b, ob):
        base = w0 + ci * CHUNK

        @plsc.parallel_loop(0, VPC, step=1)
        def _(m):
            r0 = m * LANES
            for l in range(LANES):
                r = r0 + l
                tt = lax.rem(base + r, T)
                for k in range(KD):
                    sl = pl.ds(k * LANES, LANES)
                    outb[ob][r, sl] = rows_v[b, r, sl] + pat_v[tt, sl]

    for b in range(NBUF):
        gather_start(b, b)

    def g_body(g, _):
        for b in range(NBUF):
            ci = NBUF * g + b
            ob = b % 2
            gather_wait(b)
            if b < 2:
                @pl.when(g > 0)
                def _():
                    out_wait(ci - 2, ob)
            else:
                out_wait(ci - 2, ob)
            extract_add(ci, b, ob)
            out_start(ci, ob)

            @pl.when(g < NGRP - 1)
            def _():
                gather_start(ci + NBUF, b)
        return ()

    lax.fori_loop(0, NGRP, g_body, ())
    out_wait(NCHUNK - 2, 0)
    out_wait(NCHUNK - 1, 1)


@jax.jit
def _embed(idx1d, tokp, pos):
    mesh = plsc.VectorSubcoreMesh(core_axis_name="c", subcore_axis_name="s")
    f = pl.kernel(
        _body,
        mesh=mesh,
        out_type=jax.ShapeDtypeStruct((FLAT, D), jnp.float32),
        scratch_types=[
            pltpu.VMEM((CHUNK,), jnp.int32),
            pltpu.VMEM((CHUNK,), jnp.int32),
            pltpu.VMEM((CHUNK,), jnp.int32),
            pltpu.VMEM((CHUNK,), jnp.int32),
            pltpu.VMEM((NBUF, CHUNK, 2 * D), jnp.float32),
            pltpu.VMEM((CHUNK, D), jnp.float32),
            pltpu.VMEM((CHUNK, D), jnp.float32),
            pltpu.VMEM((T, D), jnp.float32),
            pltpu.SemaphoreType.DMA,
            pltpu.SemaphoreType.DMA,
            pltpu.SemaphoreType.DMA,
            pltpu.SemaphoreType.DMA,
            pltpu.SemaphoreType.DMA,
            pltpu.SemaphoreType.DMA,
        ],
        compiler_params=pltpu.CompilerParams(use_tc_tiling_on_sc=True),
    )
    return f(idx1d, tokp, pos)


def kernel(idx, token_embedding_table, position_embedding_table):
    idx1d = idx.astype(jnp.int32).reshape(FLAT)
    tokp = jnp.pad(token_embedding_table, ((0, 0), (0, D)))
    out = _embed(idx1d, tokp, position_embedding_table)
    return out.reshape(B, T, D)
